# baseline (speedup 1.0000x reference)
import jax
import jax.numpy as jnp
from jax import lax
from jax.experimental import pallas as pl
from jax.experimental.pallas import tpu as pltpu
from jax.experimental.pallas import tpu_sc as plsc

BATCH = 1024
SEQ = 200
DIM = 64
PAD = 128
FLAT = BATCH * SEQ
LANES = 16
NC = 2
NS = 16
NW = NC * NS
CHUNKS = BATCH // NW
ROWS_PER_TILE = CHUNKS * SEQ
NBUF = 4
SPLIT = 128


def _encoder_call(idx2_flat, offs_flat, table2, pos_flat):
    mesh = plsc.VectorSubcoreMesh(core_axis_name="c", subcore_axis_name="s")

    @pl.kernel(
        out_type=jax.ShapeDtypeStruct((FLAT, PAD), jnp.float32),
        mesh=mesh,
        scratch_types=[
            pltpu.VMEM((ROWS_PER_TILE,), jnp.int32),
            pltpu.VMEM((SEQ * DIM,), jnp.float32),
            pltpu.VMEM((NBUF, SEQ, PAD), jnp.float32),
            pltpu.VMEM((SEQ + LANES,), jnp.int32),
            pltpu.VMEM((SEQ + LANES,), jnp.int32),
            pltpu.VMEM((SEQ + LANES,), jnp.int32),
            pltpu.VMEM((SEQ + LANES,), jnp.int32),
            pltpu.SemaphoreType.DMA((NBUF,)),
            pltpu.SemaphoreType.DMA((NBUF,)),
            pltpu.SemaphoreType.DMA((NBUF,)),
        ],
    )
    def enc_kernel(table_hbm, idx_hbm, offs_hbm, pos_hbm, out_hbm,
                   idx_v, pos_v, rows_v, offs_s0, offs_s1, offs_s2, offs_s3,
                   gsem, osem, ssem):
        offs_bufs = (offs_s0, offs_s1, offs_s2, offs_s3)
        wid = lax.axis_index("s") * NC + lax.axis_index("c")
        base = wid * ROWS_PER_TILE

        pltpu.sync_copy(idx_hbm.at[pl.ds(base, ROWS_PER_TILE)], idx_v)
        pltpu.sync_copy(pos_hbm, pos_v)

        def gather_descs(q, b):
            return (
                pltpu.make_async_copy(
                    table_hbm.at[idx_v.at[pl.ds(q * SEQ, SPLIT)]],
                    rows_v.at[b, pl.ds(0, SPLIT), :],
                    gsem.at[b],
                ),
                pltpu.make_async_copy(
                    table_hbm.at[idx_v.at[pl.ds(q * SEQ + SPLIT, SEQ - SPLIT)]],
                    rows_v.at[b, pl.ds(SPLIT, SEQ - SPLIT), :],
                    gsem.at[b],
                ),
            )

        def offs_desc(q, b):
            return pltpu.make_async_copy(
                offs_hbm.at[pl.ds(base + q * SEQ, SEQ)],
                offs_bufs[b].at[pl.ds(0, SEQ)],
                ssem.at[b],
            )

        def out_desc(q, b):
            return pltpu.make_async_copy(
                rows_v.at[b],
                out_hbm.at[pl.ds(base + q * SEQ, SEQ), :],
                osem.at[b],
            )

        def start_fetch(q, b):
            d1, d2 = gather_descs(q, b)
            d1.start()
            d2.start()
            offs_desc(q, b).start()

        def wait_fetch(q, b):
            d1, d2 = gather_descs(q, b)
            d1.wait()
            d2.wait()
            offs_desc(q, b).wait()

        for j in range(NBUF - 1):
            start_fetch(j, j)

        @pl.loop(0, CHUNKS, step=NBUF)
        def _(c0):
            for j in range(NBUF):
                q = c0 + j
                qpre = q + NBUF - 1
                bpre = (j + NBUF - 1) % NBUF

                @pl.when(qpre < CHUNKS)
                def _():
                    @pl.when(q >= 1)
                    def _():
                        out_desc(q - 1, bpre).wait()

                    start_fetch(qpre, bpre)

                wait_fetch(q, j)

                @pl.loop(0, SEQ)
                def _(r):
                    off = offs_bufs[j][pl.ds(r, LANES)][0]
                    for cc in range(0, DIM, LANES):
                        rows_v[j, r, pl.ds(cc, LANES)] = (
                            rows_v[j, r, pl.ds(off + cc, LANES)]
                            + pos_v[pl.ds(r * DIM + cc, LANES)]
                        )

                out_desc(q, j).start()

        for j in range(NBUF):
            out_desc(CHUNKS - NBUF + j, j).wait()

    return enc_kernel(table2, idx2_flat, offs_flat, pos_flat)


def kernel(text, embedding_weight, positional_encoding):
    seq_len = text.shape[1]
    text_ids = text.astype(jnp.int32)
    idx2_flat = (text_ids // 2).reshape(FLAT)
    offs_flat = ((text_ids % 2) * DIM).reshape(FLAT)
    pos_flat = positional_encoding[0, :seq_len, :].reshape(SEQ * DIM)
    table2 = embedding_weight.reshape(500000, PAD)
    out = _encoder_call(idx2_flat, offs_flat, table2, pos_flat)
    return out[:, :DIM].reshape(BATCH, SEQ, DIM)


# speedup vs baseline: 1.2263x; 1.2263x over previous
import jax
import jax.numpy as jnp
from jax import lax
from jax.experimental import pallas as pl
from jax.experimental.pallas import tpu as pltpu
from jax.experimental.pallas import tpu_sc as plsc

BATCH = 1024
SEQ = 200
DIM = 64
PAD = 128
FLAT = BATCH * SEQ
LANES = 16
NC = 2
NS = 16
NW = NC * NS
CHUNKS = BATCH // NW
ROWS_PER_TILE = CHUNKS * SEQ
NBUF = 4
SPLIT = 128


def _encoder_call(idx_flat, table128, pos_flat):
    mesh = plsc.VectorSubcoreMesh(core_axis_name="c", subcore_axis_name="s")

    @pl.kernel(
        out_type=jax.ShapeDtypeStruct((FLAT, PAD), jnp.float32),
        mesh=mesh,
        scratch_types=[
            pltpu.VMEM((ROWS_PER_TILE,), jnp.int32),
            pltpu.VMEM((SEQ * DIM,), jnp.float32),
            pltpu.VMEM((NBUF, SEQ, PAD), jnp.float32),
            pltpu.SemaphoreType.DMA((NBUF,)),
            pltpu.SemaphoreType.DMA((NBUF,)),
        ],
    )
    def enc_kernel(table_hbm, idx_hbm, pos_hbm, out_hbm,
                   idx_v, pos_v, rows_v, gsem, osem):
        wid = lax.axis_index("s") * NC + lax.axis_index("c")
        base = wid * ROWS_PER_TILE

        pltpu.sync_copy(idx_hbm.at[pl.ds(base, ROWS_PER_TILE)], idx_v)
        pltpu.sync_copy(pos_hbm, pos_v)

        def gather_descs(q, b):
            return (
                pltpu.make_async_copy(
                    table_hbm.at[idx_v.at[pl.ds(q * SEQ, SPLIT)]],
                    rows_v.at[b, pl.ds(0, SPLIT), :],
                    gsem.at[b],
                ),
                pltpu.make_async_copy(
                    table_hbm.at[idx_v.at[pl.ds(q * SEQ + SPLIT, SEQ - SPLIT)]],
                    rows_v.at[b, pl.ds(SPLIT, SEQ - SPLIT), :],
                    gsem.at[b],
                ),
            )

        def out_desc(q, b):
            return pltpu.make_async_copy(
                rows_v.at[b],
                out_hbm.at[pl.ds(base + q * SEQ, SEQ), :],
                osem.at[b],
            )

        def start_fetch(q, b):
            d1, d2 = gather_descs(q, b)
            d1.start()
            d2.start()

        def wait_fetch(q, b):
            d1, d2 = gather_descs(q, b)
            d1.wait()
            d2.wait()

        for j in range(NBUF - 1):
            start_fetch(j, j)

        @pl.loop(0, CHUNKS, step=NBUF)
        def _(c0):
            for j in range(NBUF):
                q = c0 + j
                qpre = q + NBUF - 1
                bpre = (j + NBUF - 1) % NBUF

                @pl.when(qpre < CHUNKS)
                def _():
                    @pl.when(q >= 1)
                    def _():
                        out_desc(q - 1, bpre).wait()

                    start_fetch(qpre, bpre)

                wait_fetch(q, j)

                @pl.loop(0, SEQ)
                def _(r):
                    for cc in range(0, DIM, LANES):
                        rows_v[j, r, pl.ds(cc, LANES)] = (
                            rows_v[j, r, pl.ds(cc, LANES)]
                            + pos_v[pl.ds(r * DIM + cc, LANES)]
                        )

                out_desc(q, j).start()

        for j in range(NBUF):
            out_desc(CHUNKS - NBUF + j, j).wait()

    return enc_kernel(table128, idx_flat, pos_flat)


def kernel(text, embedding_weight, positional_encoding):
    seq_len = text.shape[1]
    idx_flat = text.astype(jnp.int32).reshape(FLAT)
    pos_flat = positional_encoding[0, :seq_len, :].reshape(SEQ * DIM)
    table128 = jnp.pad(embedding_weight, ((0, 0), (0, PAD - DIM)))
    out = _encoder_call(idx_flat, table128, pos_flat)
    return out[:, :DIM].reshape(BATCH, SEQ, DIM)
